# Initial kernel scaffold; baseline (speedup 1.0000x reference)
import jax
import jax.numpy as jnp
from jax.experimental import pallas as pl


def kernel(edge_index, edge_weight, user_emb, item_emb):
    raise NotImplementedError("write your pallas kernel here")


# speedup vs baseline: 5.1060x; 5.1060x over previous
import dataclasses
import functools

import jax
import jax.numpy as jnp
from jax import lax
from jax.experimental import pallas as pl
from jax.experimental.pallas import tpu as pltpu
from jax.experimental.pallas import tpu_sc as plsc

N_SUBCORES = 16
LANES = 16
CHUNK = 1024
KROWS = CHUNK // 128
HALF = 16
N_LAYERS_K = 3


def _build_sc_kernel(n_nodes, total_chunks):
    chunks_per_sub = total_chunks // N_SUBCORES
    rows_per_sub = n_nodes // N_SUBCORES
    drain_rows = 224
    drain_iters = rows_per_sub // drain_rows

    mesh = plsc.VectorSubcoreMesh(core_axis_name="c", subcore_axis_name="s")
    half_t = jax.ShapeDtypeStruct((n_nodes, HALF), jnp.float32)

    cp = pltpu.CompilerParams()
    if "needs_layout_passes" in pltpu.CompilerParams.__dataclass_fields__:
        cp = dataclasses.replace(cp, needs_layout_passes=False)
    if "use_tc_tiling_on_sc" in pltpu.CompilerParams.__dataclass_fields__:
        cp = dataclasses.replace(cp, use_tc_tiling_on_sc=False)

    @functools.partial(
        pl.kernel,
        out_type=(half_t, half_t, half_t, half_t),
        mesh=mesh,
        compiler_params=cp,
        scratch_types=[
            pltpu.VMEM_SHARED((n_nodes, HALF), jnp.float32),
            pltpu.VMEM((KROWS, 128), jnp.int32),
            pltpu.VMEM((KROWS, 128), jnp.int32),
            pltpu.VMEM((CHUNK,), jnp.float32),
            pltpu.VMEM((CHUNK, HALF), jnp.float32),
            pltpu.VMEM((drain_rows, HALF), jnp.float32),
            pltpu.VMEM((drain_rows, HALF), jnp.float32),
        ],
    )
    def lightgcn_sc(zeros_hbm, col_hbm, row_hbm, w_hbm, t0, t1,
                    s0, s1, cur0, cur1,
                    acc, col_v, row_v, w_v, msg_v, a_v, b_v):
        cid = lax.axis_index("c")
        sid = lax.axis_index("s")

        def run_core(t_hbm, cur_hbm, s_hbm):
            my_rows = pl.ds(sid * rows_per_sub, rows_per_sub)
            pltpu.sync_copy(zeros_hbm, acc.at[my_rows])

            plsc.subcore_barrier()

            for layer in range(N_LAYERS_K):
                src = t_hbm if layer == 0 else cur_hbm

                @pl.loop(0, chunks_per_sub)
                def _(c):
                    ci = sid * chunks_per_sub + c
                    pltpu.sync_copy(col_hbm.at[ci], col_v)
                    pltpu.sync_copy(row_hbm.at[ci], row_v)
                    pltpu.sync_copy(w_hbm.at[ci], w_v)

                    @pl.loop(0, KROWS)
                    def _(j):
                        pltpu.sync_copy(src.at[col_v.at[j]],
                                        msg_v.at[pl.ds(j * 128, 128)])

                    @pl.loop(0, CHUNK)
                    def _(e):
                        wb = plsc.load_gather(
                            w_v, [jnp.full((LANES,), e, jnp.int32)])
                        msg_v[e, :] = msg_v[e, :] * wb

                    @pl.loop(0, KROWS)
                    def _(j):
                        pltpu.sync_copy(msg_v.at[pl.ds(j * 128, 128)],
                                        acc.at[row_v.at[j]], add=True)

                plsc.subcore_barrier()

                prev = t_hbm if layer == 0 else s_hbm

                if layer < N_LAYERS_K - 1:
                    pltpu.sync_copy(acc.at[my_rows], cur_hbm.at[my_rows])

                @pl.loop(0, drain_iters)
                def _(d):
                    base = sid * rows_per_sub + d * drain_rows
                    sl = pl.ds(base, drain_rows)
                    pltpu.sync_copy(acc.at[sl], a_v)
                    pltpu.sync_copy(prev.at[sl], b_v)

                    if layer == N_LAYERS_K - 1:
                        @pl.loop(0, drain_rows)
                        def _(i):
                            b_v[i, :] = (a_v[i, :] + b_v[i, :]) * 0.25
                    else:
                        @pl.loop(0, drain_rows)
                        def _(i):
                            b_v[i, :] = a_v[i, :] + b_v[i, :]

                    pltpu.sync_copy(b_v, s_hbm.at[sl])

                if layer < N_LAYERS_K - 1:
                    pltpu.sync_copy(zeros_hbm, acc.at[my_rows])

                plsc.subcore_barrier()

        @pl.when(cid == 0)
        def _():
            run_core(t0, cur0, s0)

        @pl.when(cid == 1)
        def _():
            run_core(t1, cur1, s1)

    return lightgcn_sc


def kernel(edge_index, edge_weight, user_emb, item_emb):
    n_users = user_emb.shape[0]
    n_nodes = n_users + item_emb.shape[0]
    n_edges = edge_weight.shape[0]

    rows_unit = N_SUBCORES * 1568
    n_nodes_pad = ((n_nodes + rows_unit - 1) // rows_unit) * rows_unit
    node_pad = n_nodes_pad - n_nodes

    edges_per_sub = CHUNK * N_SUBCORES
    e_pad = ((n_edges + edges_per_sub - 1) // edges_per_sub) * edges_per_sub
    total_chunks = e_pad // CHUNK
    pad = e_pad - n_edges

    row = edge_index[0].astype(jnp.int32)
    col = edge_index[1].astype(jnp.int32)
    w = edge_weight.astype(jnp.float32)
    if pad:
        zi = jnp.zeros((pad,), jnp.int32)
        row = jnp.concatenate([row, zi])
        col = jnp.concatenate([col, zi])
        w = jnp.concatenate([w, jnp.zeros((pad,), jnp.float32)])

    col3 = col.reshape(total_chunks, KROWS, 128)
    row3 = row.reshape(total_chunks, KROWS, 128)
    w2 = w.reshape(total_chunks, CHUNK)

    all_emb = jnp.concatenate([user_emb, item_emb], axis=0)
    if node_pad:
        all_emb = jnp.concatenate(
            [all_emb, jnp.zeros((node_pad, all_emb.shape[1]), jnp.float32)])
    t0 = all_emb[:, :HALF]
    t1 = all_emb[:, HALF:]

    zeros_hbm = jnp.zeros((n_nodes_pad // N_SUBCORES, HALF), jnp.float32)

    fn = _build_sc_kernel(n_nodes_pad, total_chunks)
    s0, s1, _, _ = fn(zeros_hbm, col3, row3, w2, t0, t1)

    final = jnp.concatenate([s0, s1], axis=1)
    return (final[:n_users], final[n_users:n_nodes])


# speedup vs baseline: 7.4941x; 1.4677x over previous
import dataclasses
import functools

import jax
import jax.numpy as jnp
from jax import lax
from jax.experimental import pallas as pl
from jax.experimental.pallas import tpu as pltpu
from jax.experimental.pallas import tpu_sc as plsc

N_SUBCORES = 16
LANES = 16
CHUNK = 512
KROWS = CHUNK // 128
HALF = 16
N_LAYERS_K = 3


def _build_sc_kernel(n_nodes, total_chunks):
    chunks_per_sub = total_chunks // N_SUBCORES
    rows_per_sub = n_nodes // N_SUBCORES
    drain_rows = 224
    drain_iters = rows_per_sub // drain_rows

    mesh = plsc.VectorSubcoreMesh(core_axis_name="c", subcore_axis_name="s")
    half_t = jax.ShapeDtypeStruct((n_nodes, HALF), jnp.float32)

    cp = pltpu.CompilerParams()
    if "needs_layout_passes" in pltpu.CompilerParams.__dataclass_fields__:
        cp = dataclasses.replace(cp, needs_layout_passes=False)
    if "use_tc_tiling_on_sc" in pltpu.CompilerParams.__dataclass_fields__:
        cp = dataclasses.replace(cp, use_tc_tiling_on_sc=False)

    @functools.partial(
        pl.kernel,
        out_type=(half_t, half_t, half_t, half_t),
        mesh=mesh,
        compiler_params=cp,
        scratch_types=[
            pltpu.VMEM_SHARED((n_nodes, HALF), jnp.float32),
            pltpu.VMEM((KROWS, 128), jnp.int32),
            pltpu.VMEM((KROWS, 128), jnp.int32),
            pltpu.VMEM((KROWS, 128), jnp.int32),
            pltpu.VMEM((KROWS, 128), jnp.int32),
            pltpu.VMEM((CHUNK,), jnp.float32),
            pltpu.VMEM((CHUNK,), jnp.float32),
            pltpu.VMEM((CHUNK, HALF), jnp.float32),
            pltpu.VMEM((CHUNK, HALF), jnp.float32),
            pltpu.VMEM((drain_rows, HALF), jnp.float32),
            pltpu.VMEM((drain_rows, HALF), jnp.float32),
            pltpu.SemaphoreType.DMA,
            pltpu.SemaphoreType.DMA,
            pltpu.SemaphoreType.DMA,
        ],
    )
    def lightgcn_sc(zeros_hbm, col_hbm, row_hbm, w_hbm, t0, t1,
                    s0, s1, cur0, cur1,
                    acc, col_v0, col_v1, row_v0, row_v1, w_v0, w_v1,
                    msg_v0, msg_v1, a_v, b_v, sem_i, sem_g, sem_s):
        cid = lax.axis_index("c")
        sid = lax.axis_index("s")
        col_s = (col_v0, col_v1)
        row_s = (row_v0, row_v1)
        w_s = (w_v0, w_v1)
        msg_s = (msg_v0, msg_v1)

        def run_core(t_hbm, cur_hbm, s_hbm):
            my_rows = pl.ds(sid * rows_per_sub, rows_per_sub)
            pltpu.sync_copy(zeros_hbm, acc.at[my_rows])

            plsc.subcore_barrier()

            first_ci = sid * chunks_per_sub

            def issue_idx(ci, b):
                pltpu.async_copy(col_hbm.at[ci], col_s[b], sem_i)
                pltpu.async_copy(row_hbm.at[ci], row_s[b], sem_i)
                pltpu.async_copy(w_hbm.at[ci], w_s[b], sem_i)

            def wait_idx(ci, b):
                pltpu.make_async_copy(col_hbm.at[ci], col_s[b], sem_i).wait()
                pltpu.make_async_copy(row_hbm.at[ci], row_s[b], sem_i).wait()
                pltpu.make_async_copy(w_hbm.at[ci], w_s[b], sem_i).wait()

            def drain_scatters(b):
                for j in range(KROWS):
                    pltpu.make_async_copy(
                        msg_s[b].at[pl.ds(j * 128, 128)],
                        acc.at[row_s[b].at[j]], sem_s).wait()

            for layer in range(N_LAYERS_K):
                src = t_hbm if layer == 0 else cur_hbm

                issue_idx(first_ci, 0)

                @pl.loop(0, chunks_per_sub, step=2)
                def _(c):
                    for b in range(2):
                        n = c + b
                        ci = first_ci + n
                        wait_idx(ci, b)

                        @pl.when(n >= 1)
                        def _():
                            drain_scatters(1 - b)

                        @pl.when(n + 1 < chunks_per_sub)
                        def _():
                            issue_idx(ci + 1, 1 - b)

                        for j in range(KROWS):
                            pltpu.async_copy(
                                src.at[col_s[b].at[j]],
                                msg_s[b].at[pl.ds(j * 128, 128)], sem_g)
                        for j in range(KROWS):
                            pltpu.make_async_copy(
                                src.at[col_s[b].at[j]],
                                msg_s[b].at[pl.ds(j * 128, 128)], sem_g).wait()

                        wv = w_s[b]
                        mv = msg_s[b]

                        @pl.loop(0, CHUNK)
                        def _(e):
                            wb = plsc.load_gather(
                                wv, [jnp.full((LANES,), e, jnp.int32)])
                            mv[e, :] = mv[e, :] * wb

                        for j in range(KROWS):
                            pltpu.async_copy(
                                msg_s[b].at[pl.ds(j * 128, 128)],
                                acc.at[row_s[b].at[j]], sem_s, add=True)

                drain_scatters((chunks_per_sub - 1) % 2)
                plsc.subcore_barrier()

                prev = t_hbm if layer == 0 else s_hbm

                if layer < N_LAYERS_K - 1:
                    pltpu.sync_copy(acc.at[my_rows], cur_hbm.at[my_rows])

                @pl.loop(0, drain_iters)
                def _(d):
                    base = sid * rows_per_sub + d * drain_rows
                    sl = pl.ds(base, drain_rows)
                    pltpu.sync_copy(acc.at[sl], a_v)
                    pltpu.sync_copy(prev.at[sl], b_v)

                    if layer == N_LAYERS_K - 1:
                        @pl.loop(0, drain_rows)
                        def _(i):
                            b_v[i, :] = (a_v[i, :] + b_v[i, :]) * 0.25
                    else:
                        @pl.loop(0, drain_rows)
                        def _(i):
                            b_v[i, :] = a_v[i, :] + b_v[i, :]

                    pltpu.sync_copy(b_v, s_hbm.at[sl])

                if layer < N_LAYERS_K - 1:
                    pltpu.sync_copy(zeros_hbm, acc.at[my_rows])

                plsc.subcore_barrier()

        @pl.when(cid == 0)
        def _():
            run_core(t0, cur0, s0)

        @pl.when(cid == 1)
        def _():
            run_core(t1, cur1, s1)

    return lightgcn_sc


def kernel(edge_index, edge_weight, user_emb, item_emb):
    n_users = user_emb.shape[0]
    n_nodes = n_users + item_emb.shape[0]
    n_edges = edge_weight.shape[0]

    rows_unit = N_SUBCORES * 1568
    n_nodes_pad = ((n_nodes + rows_unit - 1) // rows_unit) * rows_unit
    node_pad = n_nodes_pad - n_nodes

    edges_per_sub = CHUNK * N_SUBCORES
    e_pad = ((n_edges + edges_per_sub - 1) // edges_per_sub) * edges_per_sub
    total_chunks = e_pad // CHUNK
    pad = e_pad - n_edges

    row = edge_index[0].astype(jnp.int32)
    col = edge_index[1].astype(jnp.int32)
    w = edge_weight.astype(jnp.float32)
    if pad:
        zi = jnp.zeros((pad,), jnp.int32)
        row = jnp.concatenate([row, zi])
        col = jnp.concatenate([col, zi])
        w = jnp.concatenate([w, jnp.zeros((pad,), jnp.float32)])

    col3 = col.reshape(total_chunks, KROWS, 128)
    row3 = row.reshape(total_chunks, KROWS, 128)
    w2 = w.reshape(total_chunks, CHUNK)

    all_emb = jnp.concatenate([user_emb, item_emb], axis=0)
    if node_pad:
        all_emb = jnp.concatenate(
            [all_emb, jnp.zeros((node_pad, all_emb.shape[1]), jnp.float32)])
    t0 = all_emb[:, :HALF]
    t1 = all_emb[:, HALF:]

    zeros_hbm = jnp.zeros((n_nodes_pad // N_SUBCORES, HALF), jnp.float32)

    fn = _build_sc_kernel(n_nodes_pad, total_chunks)
    s0, s1, _, _ = fn(zeros_hbm, col3, row3, w2, t0, t1)

    final = jnp.concatenate([s0, s1], axis=1)
    return (final[:n_users], final[n_users:n_nodes])


# speedup vs baseline: 16.2179x; 2.1641x over previous
import dataclasses
import functools

import jax
import jax.numpy as jnp
from jax import lax
from jax.experimental import pallas as pl
from jax.experimental.pallas import tpu as pltpu
from jax.experimental.pallas import tpu_sc as plsc

N_SUBCORES = 16
LANES = 16
CHUNK = 512
KROWS = CHUNK // 128
HALF = 16
N_LAYERS_K = 3


def _build_sc_kernel(n_nodes, total_chunks):
    chunks_per_sub = total_chunks // N_SUBCORES
    rows_per_sub = n_nodes // N_SUBCORES
    drain_rows = 224
    drain_iters = rows_per_sub // drain_rows

    mesh = plsc.VectorSubcoreMesh(core_axis_name="c", subcore_axis_name="s")
    half_t = jax.ShapeDtypeStruct((n_nodes, HALF), jnp.float32)

    cp = pltpu.CompilerParams()
    if "needs_layout_passes" in pltpu.CompilerParams.__dataclass_fields__:
        cp = dataclasses.replace(cp, needs_layout_passes=False)
    if "use_tc_tiling_on_sc" in pltpu.CompilerParams.__dataclass_fields__:
        cp = dataclasses.replace(cp, use_tc_tiling_on_sc=False)

    @functools.partial(
        pl.kernel,
        out_type=(half_t, half_t, half_t, half_t),
        mesh=mesh,
        compiler_params=cp,
        scratch_types=[
            pltpu.VMEM_SHARED((n_nodes, HALF), jnp.float32),
            pltpu.VMEM((KROWS, 128), jnp.int32),
            pltpu.VMEM((KROWS, 128), jnp.int32),
            pltpu.VMEM((KROWS, 128), jnp.int32),
            pltpu.VMEM((KROWS, 128), jnp.int32),
            pltpu.VMEM((CHUNK,), jnp.float32),
            pltpu.VMEM((CHUNK,), jnp.float32),
            pltpu.VMEM((CHUNK, HALF), jnp.float32),
            pltpu.VMEM((CHUNK, HALF), jnp.float32),
            pltpu.VMEM((drain_rows, HALF), jnp.float32),
            pltpu.VMEM((drain_rows, HALF), jnp.float32),
            pltpu.SemaphoreType.DMA,
            pltpu.SemaphoreType.DMA,
            pltpu.SemaphoreType.DMA,
        ],
    )
    def lightgcn_sc(zeros_hbm, col_hbm, row_hbm, w_hbm, t0, t1,
                    s0, s1, cur0, cur1,
                    acc, col_v0, col_v1, row_v0, row_v1, w_v0, w_v1,
                    msg_v0, msg_v1, a_v, b_v, sem_i, sem_g, sem_s):
        cid = lax.axis_index("c")
        sid = lax.axis_index("s")
        col_s = (col_v0, col_v1)
        row_s = (row_v0, row_v1)
        w_s = (w_v0, w_v1)
        msg_s = (msg_v0, msg_v1)

        def run_core(t_hbm, cur_hbm, s_hbm):
            my_rows = pl.ds(sid * rows_per_sub, rows_per_sub)
            pltpu.sync_copy(zeros_hbm, acc.at[my_rows])

            plsc.subcore_barrier()

            first_ci = sid * chunks_per_sub

            def issue_idx(ci, b):
                pltpu.async_copy(col_hbm.at[ci], col_s[b], sem_i)
                pltpu.async_copy(row_hbm.at[ci], row_s[b], sem_i)
                pltpu.async_copy(w_hbm.at[ci], w_s[b], sem_i)

            def wait_idx(ci, b):
                pltpu.make_async_copy(col_hbm.at[ci], col_s[b], sem_i).wait()
                pltpu.make_async_copy(row_hbm.at[ci], row_s[b], sem_i).wait()
                pltpu.make_async_copy(w_hbm.at[ci], w_s[b], sem_i).wait()

            def drain_scatters(b):
                for j in range(KROWS):
                    pltpu.make_async_copy(
                        msg_s[b].at[pl.ds(j * 128, 128)],
                        acc.at[row_s[b].at[j]], sem_s).wait()

            for layer in range(N_LAYERS_K):
                src = t_hbm if layer == 0 else cur_hbm

                issue_idx(first_ci, 0)

                @pl.loop(0, chunks_per_sub, step=2)
                def _(c):
                    for b in range(2):
                        n = c + b
                        ci = first_ci + n
                        wait_idx(ci, b)

                        @pl.when(n >= 1)
                        def _():
                            drain_scatters(1 - b)

                        @pl.when(n + 1 < chunks_per_sub)
                        def _():
                            issue_idx(ci + 1, 1 - b)

                        for j in range(KROWS):
                            pltpu.async_copy(
                                src.at[col_s[b].at[j]],
                                msg_s[b].at[pl.ds(j * 128, 128)], sem_g)

                        wv = w_s[b]
                        mv = msg_s[b]

                        for j in range(KROWS):
                            pltpu.make_async_copy(
                                src.at[col_s[b].at[j]],
                                msg_s[b].at[pl.ds(j * 128, 128)], sem_g).wait()

                            @pl.loop(0, 128 // LANES)
                            def _(k):
                                ebase = j * 128 + k * LANES
                                wv16 = wv[pl.ds(ebase, LANES)]
                                for i in range(LANES):
                                    e = ebase + i
                                    mv[e, :] = mv[e, :] * wv16[i]

                            pltpu.async_copy(
                                msg_s[b].at[pl.ds(j * 128, 128)],
                                acc.at[row_s[b].at[j]], sem_s, add=True)

                drain_scatters((chunks_per_sub - 1) % 2)
                plsc.subcore_barrier()

                prev = t_hbm if layer == 0 else s_hbm

                if layer < N_LAYERS_K - 1:
                    pltpu.sync_copy(acc.at[my_rows], cur_hbm.at[my_rows])

                @pl.loop(0, drain_iters)
                def _(d):
                    base = sid * rows_per_sub + d * drain_rows
                    sl = pl.ds(base, drain_rows)
                    pltpu.sync_copy(acc.at[sl], a_v)
                    pltpu.sync_copy(prev.at[sl], b_v)

                    if layer == N_LAYERS_K - 1:
                        @pl.loop(0, drain_rows)
                        def _(i):
                            b_v[i, :] = (a_v[i, :] + b_v[i, :]) * 0.25
                    else:
                        @pl.loop(0, drain_rows)
                        def _(i):
                            b_v[i, :] = a_v[i, :] + b_v[i, :]

                    pltpu.sync_copy(b_v, s_hbm.at[sl])

                if layer < N_LAYERS_K - 1:
                    pltpu.sync_copy(zeros_hbm, acc.at[my_rows])

                plsc.subcore_barrier()

        @pl.when(cid == 0)
        def _():
            run_core(t0, cur0, s0)

        @pl.when(cid == 1)
        def _():
            run_core(t1, cur1, s1)

    return lightgcn_sc


def kernel(edge_index, edge_weight, user_emb, item_emb):
    n_users = user_emb.shape[0]
    n_nodes = n_users + item_emb.shape[0]
    n_edges = edge_weight.shape[0]

    rows_unit = N_SUBCORES * 1568
    n_nodes_pad = ((n_nodes + rows_unit - 1) // rows_unit) * rows_unit
    node_pad = n_nodes_pad - n_nodes

    edges_per_sub = CHUNK * N_SUBCORES
    e_pad = ((n_edges + edges_per_sub - 1) // edges_per_sub) * edges_per_sub
    total_chunks = e_pad // CHUNK
    pad = e_pad - n_edges

    row = edge_index[0].astype(jnp.int32)
    col = edge_index[1].astype(jnp.int32)
    w = edge_weight.astype(jnp.float32)
    if pad:
        zi = jnp.zeros((pad,), jnp.int32)
        row = jnp.concatenate([row, zi])
        col = jnp.concatenate([col, zi])
        w = jnp.concatenate([w, jnp.zeros((pad,), jnp.float32)])

    col3 = col.reshape(total_chunks, KROWS, 128)
    row3 = row.reshape(total_chunks, KROWS, 128)
    w2 = w.reshape(total_chunks, CHUNK)

    all_emb = jnp.concatenate([user_emb, item_emb], axis=0)
    if node_pad:
        all_emb = jnp.concatenate(
            [all_emb, jnp.zeros((node_pad, all_emb.shape[1]), jnp.float32)])
    t0 = all_emb[:, :HALF]
    t1 = all_emb[:, HALF:]

    zeros_hbm = jnp.zeros((n_nodes_pad // N_SUBCORES, HALF), jnp.float32)

    fn = _build_sc_kernel(n_nodes_pad, total_chunks)
    s0, s1, _, _ = fn(zeros_hbm, col3, row3, w2, t0, t1)

    final = jnp.concatenate([s0, s1], axis=1)
    return (final[:n_users], final[n_users:n_nodes])


# speedup vs baseline: 16.5780x; 1.0222x over previous
import dataclasses
import functools

import jax
import jax.numpy as jnp
from jax import lax
from jax.experimental import pallas as pl
from jax.experimental.pallas import tpu as pltpu
from jax.experimental.pallas import tpu_sc as plsc

N_SUBCORES = 16
LANES = 16
CHUNK = 512
KROWS = CHUNK // 128
HALF = 16
N_LAYERS_K = 3


def _build_sc_kernel(n_nodes, total_chunks):
    chunks_per_sub = total_chunks // N_SUBCORES
    rows_per_sub = n_nodes // N_SUBCORES
    drain_rows = 448
    drain_iters = rows_per_sub // drain_rows

    mesh = plsc.VectorSubcoreMesh(core_axis_name="c", subcore_axis_name="s")
    half_t = jax.ShapeDtypeStruct((n_nodes, HALF), jnp.float32)

    cp = pltpu.CompilerParams()
    if "needs_layout_passes" in pltpu.CompilerParams.__dataclass_fields__:
        cp = dataclasses.replace(cp, needs_layout_passes=False)
    if "use_tc_tiling_on_sc" in pltpu.CompilerParams.__dataclass_fields__:
        cp = dataclasses.replace(cp, use_tc_tiling_on_sc=False)

    @functools.partial(
        pl.kernel,
        out_type=(half_t, half_t, half_t, half_t, half_t, half_t),
        mesh=mesh,
        compiler_params=cp,
        scratch_types=[
            pltpu.VMEM_SHARED((n_nodes, HALF), jnp.float32),
            pltpu.VMEM((KROWS, 128), jnp.int32),
            pltpu.VMEM((KROWS, 128), jnp.int32),
            pltpu.VMEM((KROWS, 128), jnp.int32),
            pltpu.VMEM((KROWS, 128), jnp.int32),
            pltpu.VMEM((CHUNK,), jnp.float32),
            pltpu.VMEM((CHUNK,), jnp.float32),
            pltpu.VMEM((CHUNK, HALF), jnp.float32),
            pltpu.VMEM((CHUNK, HALF), jnp.float32),
            pltpu.SemaphoreType.DMA,
            pltpu.SemaphoreType.DMA,
            pltpu.SemaphoreType.DMA,
        ],
    )
    def lightgcn_sc(zeros_hbm, col_hbm, row_hbm, w_hbm, t0, t1,
                    s0, s1, c1_0, c1_1, c2_0, c2_1,
                    acc, col_v0, col_v1, row_v0, row_v1, w_v0, w_v1,
                    msg_v0, msg_v1, sem_i, sem_g, sem_s):
        cid = lax.axis_index("c")
        sid = lax.axis_index("s")
        col_s = (col_v0, col_v1)
        row_s = (row_v0, row_v1)
        w_s = (w_v0, w_v1)
        msg_s = (msg_v0, msg_v1)
        a_v = msg_v0
        b_v = msg_v1

        def run_core(t_hbm, c1_hbm, c2_hbm, s_hbm):
            my_rows = pl.ds(sid * rows_per_sub, rows_per_sub)
            pltpu.sync_copy(zeros_hbm, acc.at[my_rows])

            plsc.subcore_barrier()

            first_ci = sid * chunks_per_sub

            def issue_idx(ci, b):
                pltpu.async_copy(col_hbm.at[ci], col_s[b], sem_i)
                pltpu.async_copy(row_hbm.at[ci], row_s[b], sem_i)
                pltpu.async_copy(w_hbm.at[ci], w_s[b], sem_i)

            def wait_idx(ci, b):
                pltpu.make_async_copy(col_hbm.at[ci], col_s[b], sem_i).wait()
                pltpu.make_async_copy(row_hbm.at[ci], row_s[b], sem_i).wait()
                pltpu.make_async_copy(w_hbm.at[ci], w_s[b], sem_i).wait()

            def drain_scatters(b):
                for j in range(KROWS):
                    pltpu.make_async_copy(
                        msg_s[b].at[pl.ds(j * 128, 128)],
                        acc.at[row_s[b].at[j]], sem_s).wait()

            srcs = (t_hbm, c1_hbm, c2_hbm)

            for layer in range(N_LAYERS_K):
                src = srcs[layer]

                issue_idx(first_ci, 0)

                @pl.loop(0, chunks_per_sub, step=2)
                def _(c):
                    for b in range(2):
                        n = c + b
                        ci = first_ci + n
                        wait_idx(ci, b)

                        @pl.when(n >= 1)
                        def _():
                            drain_scatters(1 - b)

                        @pl.when(n + 1 < chunks_per_sub)
                        def _():
                            issue_idx(ci + 1, 1 - b)

                        for j in range(KROWS):
                            pltpu.async_copy(
                                src.at[col_s[b].at[j]],
                                msg_s[b].at[pl.ds(j * 128, 128)], sem_g)

                        wv = w_s[b]
                        mv = msg_s[b]

                        for j in range(KROWS):
                            pltpu.make_async_copy(
                                src.at[col_s[b].at[j]],
                                msg_s[b].at[pl.ds(j * 128, 128)], sem_g).wait()

                            @pl.loop(0, 128 // LANES)
                            def _(k):
                                ebase = j * 128 + k * LANES
                                wv16 = wv[pl.ds(ebase, LANES)]
                                for i in range(LANES):
                                    e = ebase + i
                                    mv[e, :] = mv[e, :] * wv16[i]

                            pltpu.async_copy(
                                msg_s[b].at[pl.ds(j * 128, 128)],
                                acc.at[row_s[b].at[j]], sem_s, add=True)

                drain_scatters((chunks_per_sub - 1) % 2)
                plsc.subcore_barrier()

                if layer < N_LAYERS_K - 1:
                    cur_hbm = srcs[layer + 1]
                    pltpu.sync_copy(acc.at[my_rows], cur_hbm.at[my_rows])
                    pltpu.sync_copy(zeros_hbm, acc.at[my_rows])
                    plsc.subcore_barrier()

            dsl = pl.ds(0, drain_rows)

            @pl.loop(0, drain_iters)
            def _(d):
                base = sid * rows_per_sub + d * drain_rows
                sl = pl.ds(base, drain_rows)
                pltpu.sync_copy(acc.at[sl], a_v.at[dsl])
                pltpu.sync_copy(t_hbm.at[sl], b_v.at[dsl])

                @pl.loop(0, drain_rows)
                def _(i):
                    b_v[i, :] = a_v[i, :] + b_v[i, :]

                pltpu.sync_copy(c1_hbm.at[sl], a_v.at[dsl])

                @pl.loop(0, drain_rows)
                def _(i):
                    b_v[i, :] = a_v[i, :] + b_v[i, :]

                pltpu.sync_copy(c2_hbm.at[sl], a_v.at[dsl])

                @pl.loop(0, drain_rows)
                def _(i):
                    b_v[i, :] = (a_v[i, :] + b_v[i, :]) * 0.25

                pltpu.sync_copy(b_v.at[dsl], s_hbm.at[sl])

        @pl.when(cid == 0)
        def _():
            run_core(t0, c1_0, c2_0, s0)

        @pl.when(cid == 1)
        def _():
            run_core(t1, c1_1, c2_1, s1)

    return lightgcn_sc


def kernel(edge_index, edge_weight, user_emb, item_emb):
    n_users = user_emb.shape[0]
    n_nodes = n_users + item_emb.shape[0]
    n_edges = edge_weight.shape[0]

    rows_unit = N_SUBCORES * 1568
    n_nodes_pad = ((n_nodes + rows_unit - 1) // rows_unit) * rows_unit
    node_pad = n_nodes_pad - n_nodes

    edges_per_sub = CHUNK * N_SUBCORES
    e_pad = ((n_edges + edges_per_sub - 1) // edges_per_sub) * edges_per_sub
    total_chunks = e_pad // CHUNK
    pad = e_pad - n_edges

    row = edge_index[0].astype(jnp.int32)
    col = edge_index[1].astype(jnp.int32)
    w = edge_weight.astype(jnp.float32)
    if pad:
        zi = jnp.zeros((pad,), jnp.int32)
        row = jnp.concatenate([row, zi])
        col = jnp.concatenate([col, zi])
        w = jnp.concatenate([w, jnp.zeros((pad,), jnp.float32)])

    col3 = col.reshape(total_chunks, KROWS, 128)
    row3 = row.reshape(total_chunks, KROWS, 128)
    w2 = w.reshape(total_chunks, CHUNK)

    all_emb = jnp.concatenate([user_emb, item_emb], axis=0)
    if node_pad:
        all_emb = jnp.concatenate(
            [all_emb, jnp.zeros((node_pad, all_emb.shape[1]), jnp.float32)])
    t0 = all_emb[:, :HALF]
    t1 = all_emb[:, HALF:]

    zeros_hbm = jnp.zeros((n_nodes_pad // N_SUBCORES, HALF), jnp.float32)

    fn = _build_sc_kernel(n_nodes_pad, total_chunks)
    s0, s1 = fn(zeros_hbm, col3, row3, w2, t0, t1)[:2]

    final = jnp.concatenate([s0, s1], axis=1)
    return (final[:n_users], final[n_users:n_nodes])


# speedup vs baseline: 16.9590x; 1.0230x over previous
import dataclasses
import functools

import jax
import jax.numpy as jnp
from jax import lax
from jax.experimental import pallas as pl
from jax.experimental.pallas import tpu as pltpu
from jax.experimental.pallas import tpu_sc as plsc

N_SUBCORES = 16
LANES = 16
CHUNK = 512
KROWS = CHUNK // 128
HALF = 16
N_LAYERS_K = 3


def _build_sc_kernel(n_nodes, total_chunks):
    chunks_per_sub = total_chunks // N_SUBCORES
    rows_per_sub = n_nodes // N_SUBCORES
    drain_rows = 448
    drain_iters = rows_per_sub // drain_rows

    mesh = plsc.VectorSubcoreMesh(core_axis_name="c", subcore_axis_name="s")
    half_t = jax.ShapeDtypeStruct((n_nodes, HALF), jnp.float32)

    cp = pltpu.CompilerParams()
    if "needs_layout_passes" in pltpu.CompilerParams.__dataclass_fields__:
        cp = dataclasses.replace(cp, needs_layout_passes=False)
    if "use_tc_tiling_on_sc" in pltpu.CompilerParams.__dataclass_fields__:
        cp = dataclasses.replace(cp, use_tc_tiling_on_sc=False)

    @functools.partial(
        pl.kernel,
        out_type=(half_t, half_t, half_t, half_t, half_t, half_t),
        mesh=mesh,
        compiler_params=cp,
        scratch_types=[
            pltpu.VMEM_SHARED((n_nodes, HALF), jnp.float32),
            pltpu.VMEM((KROWS, 128), jnp.int32),
            pltpu.VMEM((KROWS, 128), jnp.int32),
            pltpu.VMEM((KROWS, 128), jnp.int32),
            pltpu.VMEM((KROWS, 128), jnp.int32),
            pltpu.VMEM((KROWS, 128), jnp.int32),
            pltpu.VMEM((KROWS, 128), jnp.int32),
            pltpu.VMEM((CHUNK,), jnp.float32),
            pltpu.VMEM((CHUNK,), jnp.float32),
            pltpu.VMEM((CHUNK,), jnp.float32),
            pltpu.VMEM((CHUNK, HALF), jnp.float32),
            pltpu.VMEM((CHUNK, HALF), jnp.float32),
            pltpu.VMEM((CHUNK, HALF), jnp.float32),
            pltpu.SemaphoreType.DMA,
            pltpu.SemaphoreType.DMA,
            pltpu.SemaphoreType.DMA,
        ],
    )
    def lightgcn_sc(zeros_hbm, col_hbm, row_hbm, w_hbm, t0, t1,
                    s0, s1, c1_0, c1_1, c2_0, c2_1,
                    acc, col_v0, col_v1, col_v2, row_v0, row_v1, row_v2,
                    w_v0, w_v1, w_v2, msg_v0, msg_v1, msg_v2,
                    sem_i, sem_g, sem_s):
        cid = lax.axis_index("c")
        sid = lax.axis_index("s")
        col_s = (col_v0, col_v1, col_v2)
        row_s = (row_v0, row_v1, row_v2)
        w_s = (w_v0, w_v1, w_v2)
        msg_s = (msg_v0, msg_v1, msg_v2)
        a_v = msg_v0
        b_v = msg_v1

        def run_core(t_hbm, c1_hbm, c2_hbm, s_hbm):
            my_rows = pl.ds(sid * rows_per_sub, rows_per_sub)
            pltpu.sync_copy(zeros_hbm, acc.at[my_rows])

            plsc.subcore_barrier()

            first_ci = sid * chunks_per_sub

            def issue_idx(ci, b):
                pltpu.async_copy(col_hbm.at[ci], col_s[b], sem_i)
                pltpu.async_copy(row_hbm.at[ci], row_s[b], sem_i)
                pltpu.async_copy(w_hbm.at[ci], w_s[b], sem_i)

            def wait_idx(ci, b):
                pltpu.make_async_copy(col_hbm.at[ci], col_s[b], sem_i).wait()
                pltpu.make_async_copy(row_hbm.at[ci], row_s[b], sem_i).wait()
                pltpu.make_async_copy(w_hbm.at[ci], w_s[b], sem_i).wait()

            def drain_scatters(b):
                for j in range(KROWS):
                    pltpu.make_async_copy(
                        msg_s[b].at[pl.ds(j * 128, 128)],
                        acc.at[row_s[b].at[j]], sem_s).wait()

            srcs = (t_hbm, c1_hbm, c2_hbm)

            def issue_gathers(src, b):
                for j in range(KROWS):
                    pltpu.async_copy(
                        src.at[col_s[b].at[j]],
                        msg_s[b].at[pl.ds(j * 128, 128)], sem_g)

            for layer in range(N_LAYERS_K):
                src = srcs[layer]

                issue_idx(first_ci, 0)
                issue_idx(first_ci + 1, 1)
                wait_idx(first_ci, 0)
                issue_gathers(src, 0)

                @pl.loop(0, chunks_per_sub, step=3)
                def _(c):
                    for b in range(3):
                        n = c + b
                        ci = first_ci + n
                        bp = (b + 2) % 3
                        bn = (b + 1) % 3

                        @pl.when(n >= 1)
                        def _():
                            drain_scatters(bp)

                        @pl.when(n + 2 < chunks_per_sub)
                        def _():
                            issue_idx(ci + 2, bp)

                        @pl.when(n + 1 < chunks_per_sub)
                        def _():
                            wait_idx(ci + 1, bn)
                            issue_gathers(src, bn)

                        wv = w_s[b]
                        mv = msg_s[b]
                        for j in range(KROWS):
                            pltpu.make_async_copy(
                                src.at[col_s[b].at[j]],
                                msg_s[b].at[pl.ds(j * 128, 128)], sem_g).wait()

                            @pl.loop(0, 128 // LANES)
                            def _(k):
                                ebase = j * 128 + k * LANES
                                wv16 = wv[pl.ds(ebase, LANES)]
                                for i in range(LANES):
                                    e = ebase + i
                                    mv[e, :] = mv[e, :] * wv16[i]

                            pltpu.async_copy(
                                msg_s[b].at[pl.ds(j * 128, 128)],
                                acc.at[row_s[b].at[j]], sem_s, add=True)

                drain_scatters((chunks_per_sub - 1) % 3)
                plsc.subcore_barrier()

                if layer < N_LAYERS_K - 1:
                    cur_hbm = srcs[layer + 1]
                    pltpu.sync_copy(acc.at[my_rows], cur_hbm.at[my_rows])
                    pltpu.sync_copy(zeros_hbm, acc.at[my_rows])
                    plsc.subcore_barrier()

            dsl = pl.ds(0, drain_rows)

            @pl.loop(0, drain_iters)
            def _(d):
                base = sid * rows_per_sub + d * drain_rows
                sl = pl.ds(base, drain_rows)
                pltpu.sync_copy(acc.at[sl], a_v.at[dsl])
                pltpu.sync_copy(t_hbm.at[sl], b_v.at[dsl])

                @pl.loop(0, drain_rows)
                def _(i):
                    b_v[i, :] = a_v[i, :] + b_v[i, :]

                pltpu.sync_copy(c1_hbm.at[sl], a_v.at[dsl])

                @pl.loop(0, drain_rows)
                def _(i):
                    b_v[i, :] = a_v[i, :] + b_v[i, :]

                pltpu.sync_copy(c2_hbm.at[sl], a_v.at[dsl])

                @pl.loop(0, drain_rows)
                def _(i):
                    b_v[i, :] = (a_v[i, :] + b_v[i, :]) * 0.25

                pltpu.sync_copy(b_v.at[dsl], s_hbm.at[sl])

        @pl.when(cid == 0)
        def _():
            run_core(t0, c1_0, c2_0, s0)

        @pl.when(cid == 1)
        def _():
            run_core(t1, c1_1, c2_1, s1)

    return lightgcn_sc


def kernel(edge_index, edge_weight, user_emb, item_emb):
    n_users = user_emb.shape[0]
    n_nodes = n_users + item_emb.shape[0]
    n_edges = edge_weight.shape[0]

    rows_unit = N_SUBCORES * 1568
    n_nodes_pad = ((n_nodes + rows_unit - 1) // rows_unit) * rows_unit
    node_pad = n_nodes_pad - n_nodes

    edges_per_sub = CHUNK * N_SUBCORES
    cps = (n_edges + edges_per_sub - 1) // edges_per_sub
    cps = ((cps + 2) // 3) * 3
    e_pad = cps * edges_per_sub
    total_chunks = e_pad // CHUNK
    pad = e_pad - n_edges

    row = edge_index[0].astype(jnp.int32)
    col = edge_index[1].astype(jnp.int32)
    w = edge_weight.astype(jnp.float32)
    if pad:
        zi = jnp.zeros((pad,), jnp.int32)
        row = jnp.concatenate([row, zi])
        col = jnp.concatenate([col, zi])
        w = jnp.concatenate([w, jnp.zeros((pad,), jnp.float32)])

    col3 = col.reshape(total_chunks, KROWS, 128)
    row3 = row.reshape(total_chunks, KROWS, 128)
    w2 = w.reshape(total_chunks, CHUNK)

    all_emb = jnp.concatenate([user_emb, item_emb], axis=0)
    if node_pad:
        all_emb = jnp.concatenate(
            [all_emb, jnp.zeros((node_pad, all_emb.shape[1]), jnp.float32)])
    t0 = all_emb[:, :HALF]
    t1 = all_emb[:, HALF:]

    zeros_hbm = jnp.zeros((n_nodes_pad // N_SUBCORES, HALF), jnp.float32)

    fn = _build_sc_kernel(n_nodes_pad, total_chunks)
    s0, s1 = fn(zeros_hbm, col3, row3, w2, t0, t1)[:2]

    final = jnp.concatenate([s0, s1], axis=1)
    return (final[:n_users], final[n_users:n_nodes])
